# baseline (speedup 1.0000x reference)
import jax
import jax.numpy as jnp
from jax import lax
from jax.experimental import pallas as pl
from jax.experimental.pallas import tpu as pltpu
from jax.experimental.pallas import tpu_sc as plsc

_B, _L, _D = 1024, 200, 64
_NC, _NS = 2, 16
_NW = _NC * _NS
_SENT_W = _B // _NW
_ROWS_W = _SENT_W * _L
_GW = 128
_NG = _ROWS_W // _GW


def _embed_body(ids_hbm, len_hbm, table_hbm, out_hbm, idx_v, len_v, buf, sem):
    wid = lax.axis_index("s") * _NC + lax.axis_index("c")
    row0 = wid * _ROWS_W
    pltpu.sync_copy(ids_hbm.at[wid], idx_v)
    pltpu.sync_copy(len_hbm.at[pl.ds(wid * _SENT_W, _SENT_W)],
                    len_v.at[pl.ds(0, _SENT_W)])

    zeros16 = jnp.zeros((16,), jnp.float32)

    def lookup_len(s):
        return len_v[pl.ds(s, 16)][0]

    def zrow(r, c2):
        for k in range(4):
            buf[r, pl.ds(k * 16, 16)] = zeros16
        return c2

    def block(g, carry):
        start = g * _GW
        pltpu.async_copy(table_hbm.at[idx_v.at[g]], buf, sem).wait()
        s0 = start // _L
        e0 = jnp.minimum(_GW, (s0 + 1) * _L - start)
        len0 = lookup_len(s0)
        z0 = jnp.clip(s0 * _L + len0 - start, 0, e0)
        lax.fori_loop(z0, e0, zrow, 0)
        len1 = lookup_len(jnp.minimum(s0 + 1, _SENT_W - 1))
        z1 = jnp.minimum(e0 + len1, _GW)
        lax.fori_loop(z1, _GW, zrow, 0)
        pltpu.sync_copy(buf, out_hbm.at[pl.ds(row0 + start, _GW)])
        return carry

    lax.fori_loop(0, _NG, block, 0)


@jax.jit
def _run(ids2d, len32, table):
    mesh = plsc.VectorSubcoreMesh(core_axis_name="c", subcore_axis_name="s")
    fn = pl.kernel(
        _embed_body,
        out_type=jax.ShapeDtypeStruct((_B * _L, _D), jnp.float32),
        mesh=mesh,
        compiler_params=pltpu.CompilerParams(use_tc_tiling_on_sc=False),
        scratch_types=[
            pltpu.VMEM((_NG, _GW), jnp.int32),
            pltpu.VMEM((_SENT_W + 16,), jnp.int32),
            pltpu.VMEM((_GW, _D), jnp.float32),
            pltpu.SemaphoreType.DMA,
        ],
    )
    return fn(ids2d, len32, table)


def kernel(token_ids, lengths, table):
    ids2d = token_ids.astype(jnp.int32).reshape(_NW, _NG, _GW)
    len32 = lengths.astype(jnp.int32)
    out = _run(ids2d, len32, table)
    return (lengths.astype(jnp.int64), out.reshape(_B, _L, _D))


# speedup vs baseline: 1.0593x; 1.0593x over previous
import jax
import jax.numpy as jnp
from jax import lax
from jax.experimental import pallas as pl
from jax.experimental.pallas import tpu as pltpu
from jax.experimental.pallas import tpu_sc as plsc

_B, _L, _D = 1024, 200, 64
_NC, _NS = 2, 16
_NW = _NC * _NS
_SENT_W = _B // _NW
_ROWS_W = _SENT_W * _L
_GW = 128
_NG = _ROWS_W // _GW
_NBUF = 5
_NR = _NG // _NBUF


def _embed_body(ids_hbm, len_hbm, table_hbm, out_hbm,
                idx_v, len_v, b0, b1, b2, b3, b4, in_sems, out_sems):
    bufs = (b0, b1, b2, b3, b4)
    wid = lax.axis_index("s") * _NC + lax.axis_index("c")
    row0 = wid * _ROWS_W
    pltpu.sync_copy(ids_hbm.at[wid], idx_v)
    pltpu.sync_copy(len_hbm.at[pl.ds(wid * _SENT_W, _SENT_W)],
                    len_v.at[pl.ds(0, _SENT_W)])

    zeros16 = jnp.zeros((16,), jnp.float32)

    def lookup_len(s):
        return len_v[pl.ds(s, 16)][0]

    def zero_tails(g, buf):
        start = g * _GW
        s0 = start // _L
        e0 = jnp.minimum(_GW, (s0 + 1) * _L - start)

        def zrow(r, c2):
            for k in range(4):
                buf[r, pl.ds(k * 16, 16)] = zeros16
            return c2

        z0 = jnp.clip(s0 * _L + lookup_len(s0) - start, 0, e0)
        lax.fori_loop(z0, e0, zrow, 0)
        z1 = jnp.minimum(e0 + lookup_len(jnp.minimum(s0 + 1, _SENT_W - 1)), _GW)
        lax.fori_loop(z1, _GW, zrow, 0)

    def gather(g, b):
        pltpu.async_copy(table_hbm.at[idx_v.at[g]], bufs[b], in_sems.at[b])

    def wait_gather(b):
        pltpu.make_async_copy(table_hbm.at[idx_v.at[0]], bufs[b],
                              in_sems.at[b]).wait()

    def writeback(g, b):
        pltpu.async_copy(bufs[b], out_hbm.at[pl.ds(row0 + g * _GW, _GW)],
                         out_sems.at[b])

    def wait_writeback(b):
        pltpu.make_async_copy(bufs[b], out_hbm.at[pl.ds(row0, _GW)],
                              out_sems.at[b]).wait()

    for b in range(_NBUF):
        gather(b, b)

    def round_fn(r, c):
        g0 = r * _NBUF
        for b in range(_NBUF):
            wait_gather(b)
            zero_tails(g0 + b, bufs[b])
            writeback(g0 + b, b)

        @pl.when(r < _NR - 1)
        def _rearm():
            for b in range(_NBUF):
                wait_writeback(b)
                gather(g0 + _NBUF + b, b)

        return c

    lax.fori_loop(0, _NR, round_fn, 0)

    for b in range(_NBUF):
        wait_writeback(b)


@jax.jit
def _run(ids2d, len32, table):
    mesh = plsc.VectorSubcoreMesh(core_axis_name="c", subcore_axis_name="s")
    fn = pl.kernel(
        _embed_body,
        out_type=jax.ShapeDtypeStruct((_B * _L, _D), jnp.float32),
        mesh=mesh,
        compiler_params=pltpu.CompilerParams(use_tc_tiling_on_sc=False),
        scratch_types=[
            pltpu.VMEM((_NG, _GW), jnp.int32),
            pltpu.VMEM((_SENT_W + 16,), jnp.int32),
        ] + [pltpu.VMEM((_GW, _D), jnp.float32) for _ in range(_NBUF)] + [
            pltpu.SemaphoreType.DMA((_NBUF,)),
            pltpu.SemaphoreType.DMA((_NBUF,)),
        ],
    )
    return fn(ids2d, len32, table)


def kernel(token_ids, lengths, table):
    ids2d = token_ids.astype(jnp.int32).reshape(_NW, _NG, _GW)
    len32 = lengths.astype(jnp.int32)
    out = _run(ids2d, len32, table)
    return (lengths.astype(jnp.int64), out.reshape(_B, _L, _D))
